# initial kernel scaffold (unmeasured)
import jax
import jax.numpy as jnp
from jax import lax
from jax.experimental import pallas as pl
from jax.experimental.pallas import tpu as pltpu

N_DEV = 32


def kernel(x, w_mat):
    m_total, k_shard = x.shape
    k_total, n_out = w_mat.shape
    m_blk = m_total // N_DEV

    def body(x_ref, w_ref, out_ref, gathered_ref, send_sems, recv_sems):
        my = lax.axis_index("i")

        barrier_sem = pltpu.get_barrier_semaphore()
        for s in range(1, N_DEV):
            peer = lax.rem(my + s, N_DEV)
            pl.semaphore_signal(
                barrier_sem, inc=1,
                device_id=(peer,), device_id_type=pl.DeviceIdType.MESH,
            )
        pl.semaphore_wait(barrier_sem, N_DEV - 1)

        gathered_ref[:, pl.ds(my * k_shard, k_shard)] = (
            x_ref[pl.ds(my * m_blk, m_blk), :]
        )

        sends = []
        for s in range(1, N_DEV):
            tgt = lax.rem(my + s, N_DEV)
            rdma = pltpu.make_async_remote_copy(
                src_ref=x_ref.at[pl.ds(tgt * m_blk, m_blk), :],
                dst_ref=gathered_ref.at[:, pl.ds(my * k_shard, k_shard)],
                send_sem=send_sems.at[s],
                recv_sem=recv_sems.at[N_DEV - s],
                device_id=(tgt,),
                device_id_type=pl.DeviceIdType.MESH,
            )
            rdma.start()
            sends.append(rdma)

        for s in range(1, N_DEV):
            src = lax.rem(my + (N_DEV - s), N_DEV)
            recv = pltpu.make_async_remote_copy(
                src_ref=x_ref.at[pl.ds(0, m_blk), :],
                dst_ref=gathered_ref.at[:, pl.ds(src * k_shard, k_shard)],
                send_sem=send_sems.at[s],
                recv_sem=recv_sems.at[s],
                device_id=(src,),
                device_id_type=pl.DeviceIdType.MESH,
            )
            recv.wait_recv()

        acc = jnp.dot(
            gathered_ref[:, :], w_ref[:, :],
            preferred_element_type=jnp.float32,
        )
        c = 0.7978845608028654
        out_ref[:, :] = 0.5 * acc * (
            1.0 + jnp.tanh(c * (acc + 0.044715 * acc * acc * acc))
        )

        for rdma in sends:
            rdma.wait_send()

    return pl.pallas_call(
        body,
        out_shape=jax.ShapeDtypeStruct((m_blk, n_out), jnp.float32),
        in_specs=[
            pl.BlockSpec(memory_space=pltpu.VMEM),
            pl.BlockSpec(memory_space=pltpu.VMEM),
        ],
        out_specs=pl.BlockSpec(memory_space=pltpu.VMEM),
        scratch_shapes=[
            pltpu.VMEM((m_blk, k_total), x.dtype),
            pltpu.SemaphoreType.DMA((N_DEV,)),
            pltpu.SemaphoreType.DMA((N_DEV,)),
        ],
        compiler_params=pltpu.CompilerParams(collective_id=0),
    )(x, w_mat)


# baseline (device time: 34697 ns/iter reference)
import jax
import jax.numpy as jnp
from jax import lax
from jax.experimental import pallas as pl
from jax.experimental.pallas import tpu as pltpu

N_DEV = 32
K_CHUNKS = 8


def kernel(x, w_mat):
    m_total, k_shard = x.shape
    k_total, n_out = w_mat.shape
    m_blk = m_total // N_DEV
    kc = k_total // K_CHUNKS

    def body(x_ref, w_ref, out_ref,
             xbf_ref, gathered_ref, wbuf_ref,
             send_sems, recv_sems, copy_sems):
        my = lax.axis_index("i")

        barrier_sem = pltpu.get_barrier_semaphore()
        for s in range(1, N_DEV):
            peer = lax.rem(my + s, N_DEV)
            pl.semaphore_signal(
                barrier_sem, inc=1,
                device_id=(peer,), device_id_type=pl.DeviceIdType.MESH,
            )

        w_copies = [None] * K_CHUNKS
        w_copies[0] = pltpu.make_async_copy(
            w_ref.at[pl.ds(0, kc), :], wbuf_ref.at[0], copy_sems.at[0],
        )
        w_copies[0].start()

        xbf_ref[:, :] = x_ref[:, :].astype(jnp.bfloat16)

        pl.semaphore_wait(barrier_sem, N_DEV - 1)

        gathered_ref[:, pl.ds(my * k_shard, k_shard)] = (
            xbf_ref[pl.ds(my * m_blk, m_blk), :]
        )

        sends = []
        for s in range(1, N_DEV):
            tgt = lax.rem(my + s, N_DEV)
            rdma = pltpu.make_async_remote_copy(
                src_ref=xbf_ref.at[pl.ds(tgt * m_blk, m_blk), :],
                dst_ref=gathered_ref.at[:, pl.ds(my * k_shard, k_shard)],
                send_sem=send_sems.at[s],
                recv_sem=recv_sems.at[s],
                device_id=(tgt,),
                device_id_type=pl.DeviceIdType.MESH,
            )
            rdma.start()
            sends.append(rdma)

        for s in range(1, N_DEV):
            src = lax.rem(my + (N_DEV - s), N_DEV)
            recv = pltpu.make_async_remote_copy(
                src_ref=xbf_ref.at[pl.ds(0, m_blk), :],
                dst_ref=gathered_ref.at[:, pl.ds(src * k_shard, k_shard)],
                send_sem=send_sems.at[s],
                recv_sem=recv_sems.at[s],
                device_id=(src,),
                device_id_type=pl.DeviceIdType.MESH,
            )
            recv.wait_recv()

        acc = jnp.zeros((m_blk, n_out), dtype=jnp.float32)
        for c in range(K_CHUNKS):
            if c + 1 < K_CHUNKS:
                w_copies[c + 1] = pltpu.make_async_copy(
                    w_ref.at[pl.ds((c + 1) * kc, kc), :],
                    wbuf_ref.at[(c + 1) % 2],
                    copy_sems.at[(c + 1) % 2],
                )
                w_copies[c + 1].start()
            w_copies[c].wait()
            wbf = wbuf_ref[c % 2, :, :].astype(jnp.bfloat16)
            acc = acc + jnp.dot(
                gathered_ref[:, pl.ds(c * kc, kc)], wbf,
                preferred_element_type=jnp.float32,
            )

        g = 0.7978845608028654
        out_ref[:, :] = 0.5 * acc * (
            1.0 + jnp.tanh(g * (acc + 0.044715 * acc * acc * acc))
        )

        for rdma in sends:
            rdma.wait_send()

    return pl.pallas_call(
        body,
        out_shape=jax.ShapeDtypeStruct((m_blk, n_out), jnp.float32),
        in_specs=[
            pl.BlockSpec(memory_space=pltpu.VMEM),
            pl.BlockSpec(memory_space=pl.ANY),
        ],
        out_specs=pl.BlockSpec(memory_space=pltpu.VMEM),
        scratch_shapes=[
            pltpu.VMEM((m_total, k_shard), jnp.bfloat16),
            pltpu.VMEM((m_blk, k_total), jnp.bfloat16),
            pltpu.VMEM((2, kc, n_out), jnp.float32),
            pltpu.SemaphoreType.DMA((N_DEV,)),
            pltpu.SemaphoreType.DMA((N_DEV,)),
            pltpu.SemaphoreType.DMA((2,)),
        ],
        compiler_params=pltpu.CompilerParams(collective_id=0),
    )(x, w_mat)


# device time: 23428 ns/iter; 1.4810x vs baseline; 1.4810x over previous
import os

import jax
import jax.numpy as jnp
from jax import lax
from jax.experimental import pallas as pl
from jax.experimental.pallas import tpu as pltpu

N_DEV = 32
K_CHUNKS = 8

_KVAR = os.environ.get("KVAR", "full")
_DO_COMM = _KVAR != "nocomm"
_DO_GEMM = _KVAR != "nogemm"


def kernel(x, w_mat):
    m_total, k_shard = x.shape
    k_total, n_out = w_mat.shape
    m_blk = m_total // N_DEV
    kc = k_total // K_CHUNKS

    def body(x_ref, w_ref, out_ref,
             xbf_ref, gathered_ref, wbuf_ref,
             send_sems, recv_sems, copy_sems):
        my = lax.axis_index("i")

        with jax.named_scope("barrier_and_prefetch"):
            if _DO_COMM:
                barrier_sem = pltpu.get_barrier_semaphore()
                for s in range(1, N_DEV):
                    peer = lax.rem(my + s, N_DEV)
                    pl.semaphore_signal(
                        barrier_sem, inc=1,
                        device_id=(peer,), device_id_type=pl.DeviceIdType.MESH,
                    )

            w_copies = [None] * K_CHUNKS
            if _DO_GEMM:
                w_copies[0] = pltpu.make_async_copy(
                    w_ref.at[pl.ds(0, kc), :], wbuf_ref.at[0], copy_sems.at[0],
                )
                w_copies[0].start()

            xbf_ref[:, :] = x_ref[:, :].astype(jnp.bfloat16)

            if _DO_COMM:
                pl.semaphore_wait(barrier_sem, N_DEV - 1)

        with jax.named_scope("a2a_send"):
            gathered_ref[:, pl.ds(my * k_shard, k_shard)] = (
                xbf_ref[pl.ds(my * m_blk, m_blk), :]
            )

            sends = []
            for s in range(1, N_DEV) if _DO_COMM else ():
                tgt = lax.rem(my + s, N_DEV)
                rdma = pltpu.make_async_remote_copy(
                    src_ref=xbf_ref.at[pl.ds(tgt * m_blk, m_blk), :],
                    dst_ref=gathered_ref.at[:, pl.ds(my * k_shard, k_shard)],
                    send_sem=send_sems.at[s],
                    recv_sem=recv_sems.at[s],
                    device_id=(tgt,),
                    device_id_type=pl.DeviceIdType.MESH,
                )
                rdma.start()
                sends.append(rdma)

        with jax.named_scope("a2a_wait_recv"):
            for s in range(1, N_DEV) if _DO_COMM else ():
                src = lax.rem(my + (N_DEV - s), N_DEV)
                recv = pltpu.make_async_remote_copy(
                    src_ref=xbf_ref.at[pl.ds(0, m_blk), :],
                    dst_ref=gathered_ref.at[:, pl.ds(src * k_shard, k_shard)],
                    send_sem=send_sems.at[s],
                    recv_sem=recv_sems.at[s],
                    device_id=(src,),
                    device_id_type=pl.DeviceIdType.MESH,
                )
                recv.wait_recv()

        acc = jnp.zeros((m_blk, n_out), dtype=jnp.float32)
        for c in range(K_CHUNKS) if _DO_GEMM else ():
            with jax.named_scope(f"gemm#chunk={c}"):
                if c + 1 < K_CHUNKS:
                    w_copies[c + 1] = pltpu.make_async_copy(
                        w_ref.at[pl.ds((c + 1) * kc, kc), :],
                        wbuf_ref.at[(c + 1) % 2],
                        copy_sems.at[(c + 1) % 2],
                    )
                    w_copies[c + 1].start()
                w_copies[c].wait()
                wbf = wbuf_ref[c % 2, :, :].astype(jnp.bfloat16)
                acc = acc + jnp.dot(
                    gathered_ref[:, pl.ds(c * kc, kc)], wbf,
                    preferred_element_type=jnp.float32,
                )

        with jax.named_scope("epilogue"):
            g = 0.7978845608028654
            out_ref[:, :] = 0.5 * acc * (
                1.0 + jnp.tanh(g * (acc + 0.044715 * acc * acc * acc))
            )

            for rdma in sends:
                rdma.wait_send()

    return pl.pallas_call(
        body,
        out_shape=jax.ShapeDtypeStruct((m_blk, n_out), jnp.float32),
        in_specs=[
            pl.BlockSpec(memory_space=pltpu.VMEM),
            pl.BlockSpec(memory_space=pl.ANY),
        ],
        out_specs=pl.BlockSpec(memory_space=pltpu.VMEM),
        scratch_shapes=[
            pltpu.VMEM((m_total, k_shard), jnp.bfloat16),
            pltpu.VMEM((m_blk, k_total), jnp.bfloat16),
            pltpu.VMEM((2, kc, n_out), jnp.float32),
            pltpu.SemaphoreType.DMA((N_DEV,)),
            pltpu.SemaphoreType.DMA((N_DEV,)),
            pltpu.SemaphoreType.DMA((2,)),
        ],
        compiler_params=pltpu.CompilerParams(collective_id=0),
    )(x, w_mat)
